# Initial kernel scaffold; baseline (speedup 1.0000x reference)
import functools

import jax
import jax.numpy as jnp
from jax import lax
from jax.experimental import pallas as pl
from jax.experimental.pallas import tpu as pltpu
from jax.experimental.pallas import tpu_sc as plsc

N = 10000
E = 320000
D = 128
NS = 16
C = 80
EPW = E // NS
NCHUNK = EPW // C
RPW = 640
RCH = 80
VD = D // 16


def _round_body(nb, first, *refs):
    blocks = refs[0:nb]
    srcr = refs[nb]
    dstr = refs[nb + 1]
    p = nb + 2
    if first:
        outs = refs[p:p + 2 * nb]
        inv_in_out = refs[p + 2 * nb]
        inv_out_out = refs[p + 2 * nb + 1]
        scr = refs[p + 2 * nb + 2:]
    else:
        inv_in_hbm = refs[p]
        inv_out_hbm = refs[p + 1]
        outs = refs[p + 2:p + 2 + 2 * nb]
        scr = refs[p + 2 + 2 * nb:]
    acc, inv_sh, idx_g, idx_s, gbuf, zbuf, invb, dbuf, ones16 = scr

    c = lax.axis_index("c")
    s = lax.axis_index("s")
    rbase = s * RPW
    nrch = jnp.where(s == NS - 1, 5, 8)

    @pl.when(c == 0)
    def _():
        pltpu.sync_copy(srcr.at[s], idx_g)
        pltpu.sync_copy(dstr.at[s], idx_s)

    @pl.when(c == 1)
    def _():
        pltpu.sync_copy(dstr.at[s], idx_g)
        pltpu.sync_copy(srcr.at[s], idx_s)

    zero16 = jnp.zeros((16,), jnp.float32)
    one16 = jnp.ones((16,), jnp.float32)

    def fill_zbuf(i, carry):
        for v in range(VD):
            zbuf[i, pl.ds(v * 16, 16)] = zero16
        return carry

    lax.fori_loop(0, RCH, fill_zbuf, 0)

    def fill_ones(i, carry):
        ones16[i] = one16
        return carry

    lax.fori_loop(0, C, fill_ones, 0)

    if first:
        def fill_invb0(i, carry):
            invb[i] = zero16
            return carry

        lax.fori_loop(0, RCH, fill_invb0, 0)

        def zero16_chunk(k, carry):
            pltpu.sync_copy(invb, inv_sh.at[pl.ds(rbase + k * RCH, RCH)])
            return carry

        lax.fori_loop(0, nrch, zero16_chunk, 0)
        plsc.subcore_barrier()

        def count_chunk(k, carry):
            pltpu.sync_copy(ones16, inv_sh.at[idx_s.at[k]], add=True)
            return carry

        lax.fori_loop(0, NCHUNK, count_chunk, 0)
        plsc.subcore_barrier()

        def inv_chunk(k, carry):
            r0 = rbase + k * RCH
            pltpu.sync_copy(inv_sh.at[pl.ds(r0, RCH)], invb)

            def row(r, cc):
                invb[r] = 1.0 / jnp.maximum(invb[r], 1.0)
                return cc

            lax.fori_loop(0, RCH, row, 0)
            pltpu.sync_copy(invb, inv_sh.at[pl.ds(r0, RCH)])

            @pl.when(c == 0)
            def _():
                pltpu.sync_copy(invb, inv_in_out.at[pl.ds(r0, RCH)])

            @pl.when(c == 1)
            def _():
                pltpu.sync_copy(invb, inv_out_out.at[pl.ds(r0, RCH)])

            return carry

        lax.fori_loop(0, nrch, inv_chunk, 0)
    else:
        def load_inv(k, carry):
            r0 = rbase + k * RCH

            @pl.when(c == 0)
            def _():
                pltpu.sync_copy(inv_in_hbm.at[pl.ds(r0, RCH)], invb)

            @pl.when(c == 1)
            def _():
                pltpu.sync_copy(inv_out_hbm.at[pl.ds(r0, RCH)], invb)

            pltpu.sync_copy(invb, inv_sh.at[pl.ds(r0, RCH)])
            return carry

        lax.fori_loop(0, nrch, load_inv, 0)

    def zero_acc(k, carry):
        pltpu.sync_copy(zbuf, acc.at[pl.ds(rbase + k * RCH, RCH)])
        return carry

    lax.fori_loop(0, nrch, zero_acc, 0)
    plsc.subcore_barrier()

    for j in range(nb):
        hblk = blocks[j]

        def agg_chunk(k, carry):
            pltpu.sync_copy(hblk.at[idx_g.at[k]], gbuf)
            pltpu.sync_copy(gbuf, acc.at[idx_s.at[k]], add=True)
            return carry

        lax.fori_loop(0, NCHUNK, agg_chunk, 0)
        plsc.subcore_barrier()

        def div_chunk(k, carry):
            r0 = rbase + k * RCH
            pltpu.sync_copy(acc.at[pl.ds(r0, RCH)], dbuf)
            pltpu.sync_copy(zbuf, acc.at[pl.ds(r0, RCH)])
            pltpu.sync_copy(inv_sh.at[pl.ds(r0, RCH)], invb)

            def row(r, cc):
                iv = invb[r]
                for v in range(VD):
                    dbuf[r, pl.ds(v * 16, 16)] = dbuf[r, pl.ds(v * 16, 16)] * iv
                return cc

            lax.fori_loop(0, RCH, row, 0)

            @pl.when(c == 0)
            def _():
                pltpu.sync_copy(dbuf, outs[j].at[pl.ds(r0, RCH)])

            @pl.when(c == 1)
            def _():
                pltpu.sync_copy(dbuf, outs[nb + j].at[pl.ds(r0, RCH)])

            return carry

        lax.fori_loop(0, nrch, div_chunk, 0)
        plsc.subcore_barrier()


def _make_round(nb, first):
    f32 = jnp.float32
    out_blocks = [jax.ShapeDtypeStruct((N, D), f32) for _ in range(2 * nb)]
    if first:
        out_type = tuple(out_blocks) + (
            jax.ShapeDtypeStruct((N, 16), f32),
            jax.ShapeDtypeStruct((N, 16), f32),
        )
    else:
        out_type = tuple(out_blocks)
    scratch = (
        pltpu.VMEM_SHARED((N, D), f32),
        pltpu.VMEM_SHARED((N, 16), f32),
        pltpu.VMEM((NCHUNK, C), jnp.int32),
        pltpu.VMEM((NCHUNK, C), jnp.int32),
        pltpu.VMEM((C, D), f32),
        pltpu.VMEM((RCH, D), f32),
        pltpu.VMEM((RCH, 16), f32),
        pltpu.VMEM((RCH, D), f32),
        pltpu.VMEM((C, 16), f32),
    )
    mesh = plsc.VectorSubcoreMesh(core_axis_name="c", subcore_axis_name="s")
    return pl.kernel(
        functools.partial(_round_body, nb, first),
        out_type=out_type,
        mesh=mesh,
        scratch_types=scratch,
        name=f"sage_round_nb{nb}",
    )


_round1 = _make_round(1, True)
_round2 = _make_round(2, False)
_round3 = _make_round(4, False)

RT = 400
NB3 = 8


def _mm_body(*refs):
    xs = refs[0:NB3]
    wt = refs[NB3]
    bias = refs[NB3 + 1]
    o = refs[NB3 + 2]
    acc = jnp.broadcast_to(bias[...], (RT, D)).astype(jnp.float32)
    for j in range(NB3):
        acc = acc + jnp.dot(
            xs[j][...], wt[j * D:(j + 1) * D, :],
            preferred_element_type=jnp.float32,
        )
    o[...] = acc


def _matmul(blocks, Wt, bias):
    grid = (N // RT,)
    in_specs = (
        [pl.BlockSpec((RT, D), lambda i: (i, 0)) for _ in range(NB3)]
        + [
            pl.BlockSpec((NB3 * D, D), lambda i: (0, 0)),
            pl.BlockSpec((1, D), lambda i: (0, 0)),
        ]
    )
    return pl.pallas_call(
        _mm_body,
        grid=grid,
        in_specs=in_specs,
        out_specs=pl.BlockSpec((RT, D), lambda i: (i, 0)),
        out_shape=jax.ShapeDtypeStruct((N, D), jnp.float32),
    )(*blocks, Wt, bias)


def kernel(feature, edge_index, W, b):
    src = edge_index[0].astype(jnp.int32).reshape(NS, NCHUNK, C)
    dst = edge_index[1].astype(jnp.int32).reshape(NS, NCHUNK, C)
    b_in, b_out, inv_in, inv_out = _round1(feature, src, dst)
    r2 = _round2(b_in, b_out, src, dst, inv_in, inv_out)
    r3 = _round3(*r2, src, dst, inv_in, inv_out)
    return _matmul(r3, W.T, b.reshape(1, D))


# speedup vs baseline: 2.5376x; 2.5376x over previous
import functools

import jax
import jax.numpy as jnp
from jax import lax
from jax.experimental import pallas as pl
from jax.experimental.pallas import tpu as pltpu
from jax.experimental.pallas import tpu_sc as plsc

N = 10000
E = 320000
D = 128
NS = 16
C = 80
EPW = E // NS
NCHUNK = EPW // C
RPW = 640
VD = D // 16


def _load_idx(c, srcr, dstr, s, k, idx_gc, idx_sc):
    @pl.when(c == 0)
    def _():
        pltpu.sync_copy(srcr.at[s, k], idx_gc)
        pltpu.sync_copy(dstr.at[s, k], idx_sc)

    @pl.when(c == 1)
    def _():
        pltpu.sync_copy(dstr.at[s, k], idx_gc)
        pltpu.sync_copy(srcr.at[s, k], idx_sc)


def _round_body(nb, first, *refs):
    blocks = refs[0:nb]
    srcr = refs[nb]
    dstr = refs[nb + 1]
    p = nb + 2
    if first:
        outs = refs[p:p + 2 * nb]
        inv_in = refs[p + 2 * nb]
        inv_out = refs[p + 2 * nb + 1]
        scr = refs[p + 2 * nb + 2:]
    else:
        inv_in = refs[p]
        inv_out = refs[p + 1]
        outs = refs[p + 2:p + 2 + 2 * nb]
        scr = refs[p + 2 + 2 * nb:]
    if first:
        acc, idx_gc, idx_sc, gbuf, zbuf, invb, onesb = scr
    else:
        acc, idx_gc, idx_sc, gbuf, zbuf, invb = scr

    c = lax.axis_index("c")
    s = lax.axis_index("s")
    rbase = s * RPW
    nst = jnp.where(s == NS - 1, 25, 40)

    zero16 = jnp.zeros((16,), jnp.float32)
    one16 = jnp.ones((16,), jnp.float32)

    def fill_zbuf(i, carry):
        for v in range(VD):
            zbuf[i, pl.ds(v * 16, 16)] = zero16
        return carry

    lax.fori_loop(0, 16, fill_zbuf, 0)

    def zero_acc(m, carry):
        pltpu.sync_copy(zbuf, acc.at[pl.ds(rbase + m * 16, 16)])
        return carry

    lax.fori_loop(0, nst, zero_acc, 0)
    plsc.subcore_barrier()

    if first:
        def fill_onesb(i, carry):
            for v in range(VD):
                onesb[i, pl.ds(v * 16, 16)] = one16
            return carry

        lax.fori_loop(0, C, fill_onesb, 0)

        def count_chunk(k, carry):
            _load_idx(c, srcr, dstr, s, k, idx_gc, idx_sc)
            pltpu.sync_copy(onesb, acc.at[idx_sc], add=True)
            return carry

        lax.fori_loop(0, NCHUNK, count_chunk, 0)
        plsc.subcore_barrier()

        def inv_step(m, carry):
            r0 = rbase + m * 16
            pltpu.sync_copy(acc.at[pl.ds(r0, 16)], gbuf.at[pl.ds(0, 16)])
            pltpu.sync_copy(zbuf, acc.at[pl.ds(r0, 16)])
            for r in range(16):
                invb[r] = 1.0 / jnp.maximum(gbuf[r, pl.ds(0, 16)], 1.0)

            @pl.when(c == 0)
            def _():
                pltpu.sync_copy(invb, inv_in.at[pl.ds(r0, 16)])

            @pl.when(c == 1)
            def _():
                pltpu.sync_copy(invb, inv_out.at[pl.ds(r0, 16)])

            return carry

        lax.fori_loop(0, nst, inv_step, 0)
        plsc.subcore_barrier()

    for j in range(nb):
        hblk = blocks[j]

        def agg_chunk(k, carry):
            _load_idx(c, srcr, dstr, s, k, idx_gc, idx_sc)
            pltpu.sync_copy(hblk.at[idx_gc], gbuf)
            pltpu.sync_copy(gbuf, acc.at[idx_sc], add=True)
            return carry

        lax.fori_loop(0, NCHUNK, agg_chunk, 0)
        plsc.subcore_barrier()

        def div_step(m, carry):
            r0 = rbase + m * 16
            pltpu.sync_copy(acc.at[pl.ds(r0, 16)], gbuf.at[pl.ds(0, 16)])
            pltpu.sync_copy(zbuf, acc.at[pl.ds(r0, 16)])

            @pl.when(c == 0)
            def _():
                pltpu.sync_copy(inv_in.at[pl.ds(r0, 16)], invb)

            @pl.when(c == 1)
            def _():
                pltpu.sync_copy(inv_out.at[pl.ds(r0, 16)], invb)

            for r in range(16):
                iv = invb[r]
                for v in range(VD):
                    gbuf[r, pl.ds(v * 16, 16)] = gbuf[r, pl.ds(v * 16, 16)] * iv

            @pl.when(c == 0)
            def _():
                pltpu.sync_copy(gbuf.at[pl.ds(0, 16)], outs[j].at[pl.ds(r0, 16)])

            @pl.when(c == 1)
            def _():
                pltpu.sync_copy(
                    gbuf.at[pl.ds(0, 16)], outs[nb + j].at[pl.ds(r0, 16)])

            return carry

        lax.fori_loop(0, nst, div_step, 0)
        plsc.subcore_barrier()


def _make_round(nb, first):
    f32 = jnp.float32
    out_blocks = [jax.ShapeDtypeStruct((N, D), f32) for _ in range(2 * nb)]
    if first:
        out_type = tuple(out_blocks) + (
            jax.ShapeDtypeStruct((N, 16), f32),
            jax.ShapeDtypeStruct((N, 16), f32),
        )
    else:
        out_type = tuple(out_blocks)
    scratch = [
        pltpu.VMEM_SHARED((N, D), f32),
        pltpu.VMEM((C,), jnp.int32),
        pltpu.VMEM((C,), jnp.int32),
        pltpu.VMEM((C, D), f32),
        pltpu.VMEM((16, D), f32),
        pltpu.VMEM((16, 16), f32),
    ]
    if first:
        scratch.append(pltpu.VMEM((C, D), f32))
    mesh = plsc.VectorSubcoreMesh(core_axis_name="c", subcore_axis_name="s")
    return pl.kernel(
        functools.partial(_round_body, nb, first),
        out_type=out_type,
        mesh=mesh,
        scratch_types=tuple(scratch),
        name=f"sage_round_nb{nb}",
    )


_round1 = _make_round(1, True)
_round2 = _make_round(2, False)
_round3 = _make_round(4, False)

RT = 400
NB3 = 8


def _mm_body(*refs):
    xs = refs[0:NB3]
    wt = refs[NB3]
    bias = refs[NB3 + 1]
    o = refs[NB3 + 2]
    acc = jnp.broadcast_to(bias[...], (RT, D)).astype(jnp.float32)
    for j in range(NB3):
        acc = acc + jnp.dot(
            xs[j][...], wt[j * D:(j + 1) * D, :],
            preferred_element_type=jnp.float32,
        )
    o[...] = acc


def _matmul(blocks, Wt, bias):
    grid = (N // RT,)
    in_specs = (
        [pl.BlockSpec((RT, D), lambda i: (i, 0)) for _ in range(NB3)]
        + [
            pl.BlockSpec((NB3 * D, D), lambda i: (0, 0)),
            pl.BlockSpec((1, D), lambda i: (0, 0)),
        ]
    )
    return pl.pallas_call(
        _mm_body,
        grid=grid,
        in_specs=in_specs,
        out_specs=pl.BlockSpec((RT, D), lambda i: (i, 0)),
        out_shape=jax.ShapeDtypeStruct((N, D), jnp.float32),
    )(*blocks, Wt, bias)


def kernel(feature, edge_index, W, b):
    src = edge_index[0].astype(jnp.int32).reshape(NS, NCHUNK, C)
    dst = edge_index[1].astype(jnp.int32).reshape(NS, NCHUNK, C)
    b_in, b_out, inv_in, inv_out = _round1(feature, src, dst)
    r2 = _round2(b_in, b_out, src, dst, inv_in, inv_out)
    r3 = _round3(*r2, src, dst, inv_in, inv_out)
    return _matmul(r3, W.T, b.reshape(1, D))


# speedup vs baseline: 7.1278x; 2.8089x over previous
import functools

import jax
import jax.numpy as jnp
from jax import lax
from jax.experimental import pallas as pl
from jax.experimental.pallas import tpu as pltpu
from jax.experimental.pallas import tpu_sc as plsc

N = 10000
E = 320000
D = 128
NS = 16
C = 80
EPW = E // NS
NCHUNK = EPW // C
RPW = 640
VD = D // 16


def _round_body(nb, first, *refs):
    blocks = refs[0:nb]
    srcr = refs[nb]
    dstr = refs[nb + 1]
    p = nb + 2
    if first:
        outs = refs[p:p + 2 * nb]
        inv_in = refs[p + 2 * nb]
        inv_out = refs[p + 2 * nb + 1]
        scr = refs[p + 2 * nb + 2:]
    else:
        inv_in = refs[p]
        inv_out = refs[p + 1]
        outs = refs[p + 2:p + 2 + 2 * nb]
        scr = refs[p + 2 + 2 * nb:]
    acc, idxg, idxs, gb, zbuf, invb, sem_ig, sem_is, sem_g, sem_sc = scr

    c = lax.axis_index("c")
    s = lax.axis_index("s")
    rbase = s * RPW
    nst = jnp.where(s == NS - 1, 5, 8)

    def issue_idx_s(k, slot):
        @pl.when(c == 0)
        def _():
            pltpu.async_copy(dstr.at[s, k], idxs.at[slot], sem_is)

        @pl.when(c == 1)
        def _():
            pltpu.async_copy(srcr.at[s, k], idxs.at[slot], sem_is)

    def issue_idx_g(k, slot):
        @pl.when(c == 0)
        def _():
            pltpu.async_copy(srcr.at[s, k], idxg.at[slot], sem_ig)

        @pl.when(c == 1)
        def _():
            pltpu.async_copy(dstr.at[s, k], idxg.at[slot], sem_ig)

    def wait_ig():
        pltpu.make_async_copy(srcr.at[s, 0], idxg.at[0], sem_ig).wait()

    def wait_is():
        pltpu.make_async_copy(srcr.at[s, 0], idxs.at[0], sem_is).wait()

    def wait_sc():
        pltpu.make_async_copy(gb.at[0], acc.at[idxs.at[0]], sem_sc).wait()

    zero16 = jnp.zeros((16,), jnp.float32)

    def fill_zbuf(i, carry):
        for v in range(VD):
            zbuf[i, pl.ds(v * 16, 16)] = zero16
        return carry

    lax.fori_loop(0, C, fill_zbuf, 0)

    def zero_acc(m, carry):
        pltpu.sync_copy(zbuf, acc.at[pl.ds(rbase + m * C, C)])
        return carry

    lax.fori_loop(0, nst, zero_acc, 0)
    plsc.subcore_barrier()

    if first:
        one16 = jnp.ones((16,), jnp.float32)

        def fill_ones(i, carry):
            for v in range(VD):
                gb[1, i, pl.ds(v * 16, 16)] = one16
            return carry

        lax.fori_loop(0, C, fill_ones, 0)

        issue_idx_s(0, 0)
        issue_idx_s(1, 1)

        def count_chunk(k, carry):
            @pl.when(k >= 1)
            def _():
                wait_sc()

            @pl.when(k + 2 < NCHUNK)
            def _():
                issue_idx_s(k + 2, lax.rem(k + 2, 3))

            wait_is()
            pltpu.async_copy(
                gb.at[1], acc.at[idxs.at[lax.rem(k, 3)]], sem_sc, add=True)
            return carry

        lax.fori_loop(0, NCHUNK, count_chunk, 0)
        wait_sc()
        plsc.subcore_barrier()

        def inv_step(m, carry):
            r0 = rbase + m * C
            pltpu.sync_copy(acc.at[pl.ds(r0, C)], gb.at[0])
            pltpu.sync_copy(zbuf, acc.at[pl.ds(r0, C)])
            for r in range(C):
                invb[r] = 1.0 / jnp.maximum(gb[0, r, pl.ds(0, 16)], 1.0)

            @pl.when(c == 0)
            def _():
                pltpu.sync_copy(invb, inv_in.at[pl.ds(r0, C)])

            @pl.when(c == 1)
            def _():
                pltpu.sync_copy(invb, inv_out.at[pl.ds(r0, C)])

            return carry

        lax.fori_loop(0, nst, inv_step, 0)
        plsc.subcore_barrier()

    for j in range(nb):
        hblk = blocks[j]

        def wait_g(hb=hblk):
            pltpu.make_async_copy(
                hb.at[idxg.at[0]], gb.at[0], sem_g).wait()

        def issue_gather(buf, slot, hb=hblk):
            pltpu.async_copy(hb.at[idxg.at[slot]], gb.at[buf], sem_g)

        issue_idx_g(0, 0)
        issue_idx_s(0, 0)
        issue_idx_g(1, 1)
        issue_idx_s(1, 1)
        wait_ig()
        issue_gather(0, 0)

        def agg_chunk(k, carry, wait_g=wait_g, issue_gather=issue_gather):
            cur = lax.rem(k, 2)
            nxt = 1 - cur

            @pl.when(k >= 1)
            def _():
                wait_sc()

            @pl.when(k + 2 < NCHUNK)
            def _():
                slot = lax.rem(k + 2, 3)
                issue_idx_g(k + 2, slot)
                issue_idx_s(k + 2, slot)

            @pl.when(k + 1 < NCHUNK)
            def _():
                wait_ig()
                issue_gather(nxt, lax.rem(k + 1, 3))

            wait_g()
            wait_is()
            pltpu.async_copy(
                gb.at[cur], acc.at[idxs.at[lax.rem(k, 3)]], sem_sc, add=True)
            return carry

        lax.fori_loop(0, NCHUNK, agg_chunk, 0)
        wait_sc()
        plsc.subcore_barrier()

        def div_step(m, carry, j=j):
            r0 = rbase + m * C
            pltpu.sync_copy(acc.at[pl.ds(r0, C)], gb.at[0])
            pltpu.sync_copy(zbuf, acc.at[pl.ds(r0, C)])

            @pl.when(c == 0)
            def _():
                pltpu.sync_copy(inv_in.at[pl.ds(r0, C)], invb)

            @pl.when(c == 1)
            def _():
                pltpu.sync_copy(inv_out.at[pl.ds(r0, C)], invb)

            def rowmul(r, cc):
                iv = invb[r]
                for v in range(VD):
                    gb[0, r, pl.ds(v * 16, 16)] = gb[0, r, pl.ds(v * 16, 16)] * iv
                return cc

            lax.fori_loop(0, C, rowmul, 0)

            @pl.when(c == 0)
            def _():
                pltpu.sync_copy(gb.at[0], outs[j].at[pl.ds(r0, C)])

            @pl.when(c == 1)
            def _():
                pltpu.sync_copy(gb.at[0], outs[nb + j].at[pl.ds(r0, C)])

            return carry

        lax.fori_loop(0, nst, div_step, 0)
        plsc.subcore_barrier()


def _make_round(nb, first):
    f32 = jnp.float32
    out_blocks = [jax.ShapeDtypeStruct((N, D), f32) for _ in range(2 * nb)]
    if first:
        out_type = tuple(out_blocks) + (
            jax.ShapeDtypeStruct((N, 16), f32),
            jax.ShapeDtypeStruct((N, 16), f32),
        )
    else:
        out_type = tuple(out_blocks)
    scratch = (
        pltpu.VMEM_SHARED((N, D), f32),
        pltpu.VMEM((3, C), jnp.int32),
        pltpu.VMEM((3, C), jnp.int32),
        pltpu.VMEM((2, C, D), f32),
        pltpu.VMEM((C, D), f32),
        pltpu.VMEM((C, 16), f32),
        pltpu.SemaphoreType.DMA,
        pltpu.SemaphoreType.DMA,
        pltpu.SemaphoreType.DMA,
        pltpu.SemaphoreType.DMA,
    )
    mesh = plsc.VectorSubcoreMesh(core_axis_name="c", subcore_axis_name="s")
    return pl.kernel(
        functools.partial(_round_body, nb, first),
        out_type=out_type,
        mesh=mesh,
        scratch_types=scratch,
        name=f"sage_round_nb{nb}",
    )


_round1 = _make_round(1, True)
_round2 = _make_round(2, False)
_round3 = _make_round(4, False)

RT = 400
NB3 = 8


def _mm_body(*refs):
    xs = refs[0:NB3]
    wt = refs[NB3]
    bias = refs[NB3 + 1]
    o = refs[NB3 + 2]
    acc = jnp.broadcast_to(bias[...], (RT, D)).astype(jnp.float32)
    for j in range(NB3):
        acc = acc + jnp.dot(
            xs[j][...], wt[j * D:(j + 1) * D, :],
            preferred_element_type=jnp.float32,
        )
    o[...] = acc


def _matmul(blocks, Wt, bias):
    grid = (N // RT,)
    in_specs = (
        [pl.BlockSpec((RT, D), lambda i: (i, 0)) for _ in range(NB3)]
        + [
            pl.BlockSpec((NB3 * D, D), lambda i: (0, 0)),
            pl.BlockSpec((1, D), lambda i: (0, 0)),
        ]
    )
    return pl.pallas_call(
        _mm_body,
        grid=grid,
        in_specs=in_specs,
        out_specs=pl.BlockSpec((RT, D), lambda i: (i, 0)),
        out_shape=jax.ShapeDtypeStruct((N, D), jnp.float32),
    )(*blocks, Wt, bias)


def kernel(feature, edge_index, W, b):
    src = edge_index[0].astype(jnp.int32).reshape(NS, NCHUNK, C)
    dst = edge_index[1].astype(jnp.int32).reshape(NS, NCHUNK, C)
    b_in, b_out, inv_in, inv_out = _round1(feature, src, dst)
    r2 = _round2(b_in, b_out, src, dst, inv_in, inv_out)
    r3 = _round3(*r2, src, dst, inv_in, inv_out)
    return _matmul(r3, W.T, b.reshape(1, D))
